# baseline (speedup 1.0000x reference)
import jax
import jax.numpy as jnp
from jax import lax
from jax.experimental import pallas as pl
from jax.experimental.pallas import tpu as pltpu
from jax.experimental.pallas import tpu_sc as plsc

_N = 1024
_HD = 64
_HEADS = 4
_OC = 16
_NL = 3
_E_PAD = 36864
_EPT = 2304
_CHUNKS = _EPT // 128
_ROWS_PER_SC = _N // 2
_ZELEMS = 32 * _N
_C_ELEMS = _N * _N
_C_SP_ELEMS = (_ROWS_PER_SC + 1) * _N
_PAD_IDX = _ROWS_PER_SC * _N


def _sc_count_body(src_hbm, dst_hbm, zeros_hbm, out_hbm,
                   edge_s, edge_d, idx_buf, ones_v, c_sp, zsem, esem, ssem):
    cid = lax.axis_index("c")
    sid = lax.axis_index("s")
    base = sid * _EPT

    zh = pltpu.async_copy(zeros_hbm,
                          c_sp.at[pl.ds(sid * _ZELEMS, _ZELEMS)], zsem)
    eh1 = pltpu.async_copy(src_hbm.at[pl.ds(base, _EPT)], edge_s, esem)
    eh2 = pltpu.async_copy(dst_hbm.at[pl.ds(base, _EPT)], edge_d, esem)

    for j in range(8):
        ones_v[pl.ds(j * 16, 16)] = jnp.ones((16,), jnp.float32)

    eh1.wait()
    eh2.wait()
    row0 = cid * _ROWS_PER_SC
    for g in range(_EPT // 16):
        s16 = edge_s[pl.ds(g * 16, 16)]
        d16 = edge_d[pl.ds(g * 16, 16)]
        ld = d16 - row0
        valid = (ld >= 0) & (ld < _ROWS_PER_SC)
        idx_buf[g // 8, pl.ds((g % 8) * 16, 16)] = (
            jnp.where(valid, ld * _N + s16, _PAD_IDX))

    zh.wait()
    plsc.subcore_barrier()
    hs = [pltpu.async_copy(ones_v, c_sp.at[idx_buf.at[j]], ssem, add=True)
          for j in range(_CHUNKS)]
    for h in hs:
        h.wait()
    plsc.subcore_barrier()

    pltpu.sync_copy(c_sp.at[pl.ds(sid * _ZELEMS, _ZELEMS)],
                    out_hbm.at[pl.ds((cid * 16 + sid) * _ZELEMS, _ZELEMS)])


def _sc_count(src_p, dst_p, zeros32):
    return pl.kernel(
        _sc_count_body,
        out_type=jax.ShapeDtypeStruct((_C_ELEMS,), jnp.float32),
        mesh=plsc.VectorSubcoreMesh(core_axis_name="c", subcore_axis_name="s"),
        scratch_types=[
            pltpu.VMEM((_EPT,), jnp.int32),
            pltpu.VMEM((_EPT,), jnp.int32),
            pltpu.VMEM((_CHUNKS, 128), jnp.int32),
            pltpu.VMEM((128,), jnp.float32),
            pltpu.VMEM_SHARED((_C_SP_ELEMS,), jnp.float32),
            pltpu.SemaphoreType.DMA,
            pltpu.SemaphoreType.DMA,
            pltpu.SemaphoreType.DMA,
        ],
    )(src_p, dst_p, zeros32)


def _mm(a, b):
    return jnp.matmul(a, b, precision=lax.Precision.HIGHEST)


def _ln(x, g, b):
    m = jnp.mean(x, axis=1, keepdims=True)
    v = jnp.mean((x - m) * (x - m), axis=1, keepdims=True)
    return (x - m) / jnp.sqrt(v + 1e-5) * g + b


def _tc_body(*refs):
    (c_in, rf, enc_w, enc_b, enc_g, enc_beta,
     w0, as0, ad0, b0, g0, be0,
     w1, as1, ad1, b1, g1, be1,
     w2, as2, ad2, b2, g2, be2,
     out_w, out_b,
     pw1a, pw1b, pb1, pw2, pb2,
     vw1a, vw1b, vb1, vw2, vb2,
     probs_o, val_o) = refs

    C = c_in[...]

    x = _mm(rf[...], enc_w[...]) + enc_b[...]
    x = jnp.maximum(_ln(x, enc_g[...], enc_beta[...]), 0.0)

    layers = ((w0, as0, ad0, b0, g0, be0),
              (w1, as1, ad1, b1, g1, be1),
              (w2, as2, ad2, b2, g2, be2))
    for (w, asm_w, adm_w, b, g, be) in layers:
        h = _mm(x, w[...])
        asm = _mm(h, asm_w[...])
        adm = _mm(h, adm_w[...])
        asT = lax.dot_general(asm_w[...], h, (((0,), (1,)), ((), ())),
                              precision=lax.Precision.HIGHEST)
        maxs_row = jnp.max(asm, axis=0, keepdims=True)
        maxs_col = jnp.max(asT, axis=1, keepdims=True)
        z = adm + maxs_row
        off = jnp.maximum(z, 0.2 * z)
        PA = jnp.exp(z - off)
        PB = jnp.exp(0.2 * z - off)
        QA = jnp.exp(asT - maxs_col)
        QB = jnp.exp(0.2 * (asT - maxs_col))
        ones_col = jnp.ones((_N, 1), jnp.float32)
        heads = []
        for hd in range(_HEADS):
            mt = jnp.maximum(PA[:, hd:hd + 1] * QA[hd:hd + 1, :],
                             PB[:, hd:hd + 1] * QB[hd:hd + 1, :])
            M = C * mt
            he = jnp.concatenate([h[:, hd * _OC:(hd + 1) * _OC], ones_col], 1)
            num_ext = _mm(M, he)
            heads.append(num_ext[:, :_OC] / num_ext[:, _OC:_OC + 1])
        gat = jnp.concatenate(heads, axis=1) + b[...]
        gat = jnp.maximum(_ln(gat, g[...], be[...]), 0.0)
        x = gat + x

    emb = _mm(x, out_w[...]) + out_b[...]
    em = jnp.mean(emb, axis=0, keepdims=True)

    h1 = jnp.maximum(_mm(emb, pw1a[...]) + _mm(em, pw1b[...]) + pb1[...], 0.0)
    logits = _mm(h1, pw2[...]) + pb2[...]
    lmax = jnp.max(logits, axis=1, keepdims=True)
    el = jnp.exp(logits - lmax)
    probs_o[...] = el / jnp.sum(el, axis=1, keepdims=True)

    h2 = jnp.maximum(_mm(emb, vw1a[...]) + _mm(em, vw1b[...]) + vb1[...], 0.0)
    val_o[...] = _mm(h2, vw2[...]) + vb2[...]


def _tc_call(*args):
    return pl.pallas_call(
        _tc_body,
        out_shape=(jax.ShapeDtypeStruct((_N, 5), jnp.float32),
                   jax.ShapeDtypeStruct((_N, 1), jnp.float32)),
    )(*args)


def _head_mat(a):
    rows = jnp.arange(_HD)
    return jnp.zeros((_HD, _HEADS), jnp.float32).at[rows, rows // _OC].set(
        a.reshape(-1).astype(jnp.float32))


def kernel(robot_features, edge_index, params):
    p = params
    ei = edge_index.astype(jnp.int32)
    loop = jnp.arange(_N, dtype=jnp.int32)
    src = jnp.concatenate([ei[0], loop])
    dst = jnp.concatenate([ei[1], loop])
    pad = _E_PAD - src.shape[0]
    src_p = jnp.concatenate([src, jnp.zeros((pad,), jnp.int32)])
    dst_p = jnp.concatenate([dst, jnp.full((pad,), _N, jnp.int32)])
    zeros32 = jnp.zeros((_ZELEMS,), jnp.float32)

    c = _sc_count(src_p, dst_p, zeros32).reshape(_N, _N)

    rf = jnp.pad(robot_features.astype(jnp.float32), ((0, 0), (0, 2)))
    enc_w = jnp.pad(p['enc_W'].astype(jnp.float32), ((0, 2), (0, 0)))

    args = [c, rf, enc_w, p['enc_b'], p['enc_g'], p['enc_beta']]
    for i in range(_NL):
        args += [p[f'gat{i}_W'], _head_mat(p[f'gat{i}_asrc']),
                 _head_mat(p[f'gat{i}_adst']), p[f'gat{i}_b'],
                 p[f'ln{i}_g'], p[f'ln{i}_b']]
    args += [p['out_W'], p['out_b'],
             p['pol_W1'][:_HD], p['pol_W1'][_HD:], p['pol_b1'],
             p['pol_W2'], p['pol_b2'],
             p['val_W1'][:_HD], p['val_W1'][_HD:], p['val_b1'],
             p['val_W2'], p['val_b2']]

    probs, val = _tc_call(*args)
    return (probs, val)


# speedup vs baseline: 1.5430x; 1.5430x over previous
import jax
import jax.numpy as jnp
from jax import lax
from jax.experimental import pallas as pl
from jax.experimental.pallas import tpu as pltpu
from jax.experimental.pallas import tpu_sc as plsc

_N = 1024
_HD = 64
_HEADS = 4
_OC = 16
_NL = 3
_E_PAD = 36864
_EPT = 2304
_CHUNKS = _EPT // 128
_ROWS_PER_SC = _N // 2
_ZELEMS = 32 * _N
_C_ELEMS = _N * _N
_C_SP_ELEMS = (_ROWS_PER_SC + 1) * _N
_PAD_IDX = _ROWS_PER_SC * _N


def _sc_count_body(src_hbm, dst_hbm, zeros_hbm, out_hbm,
                   edge_s, edge_d, idx_buf, ones_v, c_sp, zsem, esem, ssem):
    cid = lax.axis_index("c")
    sid = lax.axis_index("s")
    base = sid * _EPT

    zh = pltpu.async_copy(zeros_hbm,
                          c_sp.at[pl.ds(sid * _ZELEMS, _ZELEMS)], zsem)
    eh1 = pltpu.async_copy(src_hbm.at[pl.ds(base, _EPT)], edge_s, esem)
    eh2 = pltpu.async_copy(dst_hbm.at[pl.ds(base, _EPT)], edge_d, esem)

    for j in range(8):
        ones_v[pl.ds(j * 16, 16)] = jnp.ones((16,), jnp.float32)

    eh1.wait()
    eh2.wait()
    row0 = cid * _ROWS_PER_SC
    for g in range(_EPT // 16):
        s16 = edge_s[pl.ds(g * 16, 16)]
        d16 = edge_d[pl.ds(g * 16, 16)]
        ld = d16 - row0
        valid = (ld >= 0) & (ld < _ROWS_PER_SC)
        idx_buf[g // 8, pl.ds((g % 8) * 16, 16)] = (
            jnp.where(valid, ld * _N + s16, _PAD_IDX))

    zh.wait()
    plsc.subcore_barrier()
    hs = [pltpu.async_copy(ones_v, c_sp.at[idx_buf.at[j]], ssem, add=True)
          for j in range(_CHUNKS)]
    for h in hs:
        h.wait()
    plsc.subcore_barrier()

    pltpu.sync_copy(c_sp.at[pl.ds(sid * _ZELEMS, _ZELEMS)],
                    out_hbm.at[pl.ds((cid * 16 + sid) * _ZELEMS, _ZELEMS)])


def _sc_count(src_p, dst_p, zeros32):
    return pl.kernel(
        _sc_count_body,
        out_type=jax.ShapeDtypeStruct((_C_ELEMS,), jnp.float32),
        mesh=plsc.VectorSubcoreMesh(core_axis_name="c", subcore_axis_name="s"),
        scratch_types=[
            pltpu.VMEM((_EPT,), jnp.int32),
            pltpu.VMEM((_EPT,), jnp.int32),
            pltpu.VMEM((_CHUNKS, 128), jnp.int32),
            pltpu.VMEM((128,), jnp.float32),
            pltpu.VMEM_SHARED((_C_SP_ELEMS,), jnp.float32),
            pltpu.SemaphoreType.DMA,
            pltpu.SemaphoreType.DMA,
            pltpu.SemaphoreType.DMA,
        ],
    )(src_p, dst_p, zeros32)


def _mm(a, b):
    return jnp.matmul(a, b, precision=lax.Precision.DEFAULT)


def _ln(x, g, b):
    m = jnp.mean(x, axis=1, keepdims=True)
    v = jnp.mean((x - m) * (x - m), axis=1, keepdims=True)
    return (x - m) / jnp.sqrt(v + 1e-5) * g + b


def _tc_body(*refs):
    (c_in, rf, enc_w, enc_b, enc_g, enc_beta,
     w0, as0, ad0, b0, g0, be0,
     w1, as1, ad1, b1, g1, be1,
     w2, as2, ad2, b2, g2, be2,
     out_w, out_b,
     pw1a, pw1b, pb1, pw2, pb2,
     vw1a, vw1b, vb1, vw2, vb2,
     probs_o, val_o) = refs

    C = c_in[...]

    x = _mm(rf[...], enc_w[...]) + enc_b[...]
    x = jnp.maximum(_ln(x, enc_g[...], enc_beta[...]), 0.0)

    layers = ((w0, as0, ad0, b0, g0, be0),
              (w1, as1, ad1, b1, g1, be1),
              (w2, as2, ad2, b2, g2, be2))
    for (w, asm_w, adm_w, b, g, be) in layers:
        h = _mm(x, w[...])
        asm = _mm(h, asm_w[...])
        adm = _mm(h, adm_w[...])
        asT = lax.dot_general(asm_w[...], h, (((0,), (1,)), ((), ())),
                              precision=lax.Precision.DEFAULT)
        maxs_row = jnp.max(asm, axis=0, keepdims=True)
        maxs_col = jnp.max(asT, axis=1, keepdims=True)
        z = adm + maxs_row
        off = jnp.maximum(z, 0.2 * z)
        PA = jnp.exp(z - off)
        PB = jnp.exp(0.2 * z - off)
        QA = jnp.exp(asT - maxs_col)
        QB = jnp.exp(0.2 * (asT - maxs_col))
        ones_col = jnp.ones((_N, 1), jnp.float32)
        heads = []
        for hd in range(_HEADS):
            mt = jnp.maximum(PA[:, hd:hd + 1] * QA[hd:hd + 1, :],
                             PB[:, hd:hd + 1] * QB[hd:hd + 1, :])
            M = C * mt
            he = jnp.concatenate([h[:, hd * _OC:(hd + 1) * _OC], ones_col], 1)
            num_ext = _mm(M, he)
            heads.append(num_ext[:, :_OC] / num_ext[:, _OC:_OC + 1])
        gat = jnp.concatenate(heads, axis=1) + b[...]
        gat = jnp.maximum(_ln(gat, g[...], be[...]), 0.0)
        x = gat + x

    emb = _mm(x, out_w[...]) + out_b[...]
    em = jnp.mean(emb, axis=0, keepdims=True)

    h1 = jnp.maximum(_mm(emb, pw1a[...]) + _mm(em, pw1b[...]) + pb1[...], 0.0)
    logits = _mm(h1, pw2[...]) + pb2[...]
    lmax = jnp.max(logits, axis=1, keepdims=True)
    el = jnp.exp(logits - lmax)
    probs_o[...] = el / jnp.sum(el, axis=1, keepdims=True)

    h2 = jnp.maximum(_mm(emb, vw1a[...]) + _mm(em, vw1b[...]) + vb1[...], 0.0)
    val_o[...] = _mm(h2, vw2[...]) + vb2[...]


def _tc_call(*args):
    return pl.pallas_call(
        _tc_body,
        out_shape=(jax.ShapeDtypeStruct((_N, 5), jnp.float32),
                   jax.ShapeDtypeStruct((_N, 1), jnp.float32)),
    )(*args)


def _head_mat(a):
    rows = jnp.arange(_HD)
    return jnp.zeros((_HD, _HEADS), jnp.float32).at[rows, rows // _OC].set(
        a.reshape(-1).astype(jnp.float32))


def kernel(robot_features, edge_index, params):
    p = params
    ei = edge_index.astype(jnp.int32)
    loop = jnp.arange(_N, dtype=jnp.int32)
    src = jnp.concatenate([ei[0], loop])
    dst = jnp.concatenate([ei[1], loop])
    pad = _E_PAD - src.shape[0]
    src_p = jnp.concatenate([src, jnp.zeros((pad,), jnp.int32)])
    dst_p = jnp.concatenate([dst, jnp.full((pad,), _N, jnp.int32)])
    zeros32 = jnp.zeros((_ZELEMS,), jnp.float32)

    c = _sc_count(src_p, dst_p, zeros32).reshape(_N, _N)

    rf = jnp.pad(robot_features.astype(jnp.float32), ((0, 0), (0, 2)))
    enc_w = jnp.pad(p['enc_W'].astype(jnp.float32), ((0, 2), (0, 0)))

    args = [c, rf, enc_w, p['enc_b'], p['enc_g'], p['enc_beta']]
    for i in range(_NL):
        args += [p[f'gat{i}_W'], _head_mat(p[f'gat{i}_asrc']),
                 _head_mat(p[f'gat{i}_adst']), p[f'gat{i}_b'],
                 p[f'ln{i}_g'], p[f'ln{i}_b']]
    args += [p['out_W'], p['out_b'],
             p['pol_W1'][:_HD], p['pol_W1'][_HD:], p['pol_b1'],
             p['pol_W2'], p['pol_b2'],
             p['val_W1'][:_HD], p['val_W1'][_HD:], p['val_b1'],
             p['val_W2'], p['val_b2']]

    probs, val = _tc_call(*args)
    return (probs, val)
